# baseline (speedup 1.0000x reference)
import jax
import jax.numpy as jnp
from jax.experimental import pallas as pl
from jax.experimental.pallas import tpu as pltpu

N0, N1 = 64, 64
D0, D1 = 1024, 1024


def _body(pid_ref, w0_ref, w1_ref, out_ref):
    out_ref[0, :, :D0] = w0_ref[...]
    out_ref[0, :, D0:] = jnp.broadcast_to(w1_ref[0], (N0, D1))


def kernel(position_ids, w0, w1):
    B = position_ids.size
    pid = position_ids.reshape(-1).astype(jnp.int32)
    w0_2d = w0.reshape(N0, D0)
    w1_3d = w1.reshape(N1, 1, D1)

    grid_spec = pltpu.PrefetchScalarGridSpec(
        num_scalar_prefetch=1,
        grid=(B,),
        in_specs=[
            pl.BlockSpec((N0, D0), lambda i, pid_ref: (0, 0)),
            pl.BlockSpec((1, 1, D1), lambda i, pid_ref: (pid_ref[i], 0, 0)),
        ],
        out_specs=pl.BlockSpec((1, N0, D0 + D1), lambda i, pid_ref: (i, 0, 0)),
    )
    out = pl.pallas_call(
        _body,
        grid_spec=grid_spec,
        out_shape=jax.ShapeDtypeStruct((B, N0, D0 + D1), jnp.float32),
    )(pid, w0_2d, w1_3d)
    return out.reshape(*position_ids.shape, N0, D0 + D1)


# speedup vs baseline: 2.3323x; 2.3323x over previous
import jax
import jax.numpy as jnp
from jax.experimental import pallas as pl
from jax.experimental.pallas import tpu as pltpu

N0, N1 = 64, 64
D0, D1 = 1024, 1024
G = 4


def _body(pid_ref, w0_ref, *refs):
    w1_refs, out_ref = refs[:-1], refs[-1]
    for g in range(G):
        out_ref[g, :, :D0] = w0_ref[...]
        out_ref[g, :, D0:] = jnp.broadcast_to(w1_refs[g][0], (N0, D1))


def kernel(position_ids, w0, w1):
    B = position_ids.size
    pid = position_ids.reshape(-1).astype(jnp.int32)
    w0_2d = w0.reshape(N0, D0)
    w1_3d = w1.reshape(N1, 1, D1)

    def w1_map(g):
        return lambda i, pid_ref: (pid_ref[i * G + g], 0, 0)

    grid_spec = pltpu.PrefetchScalarGridSpec(
        num_scalar_prefetch=1,
        grid=(B // G,),
        in_specs=[pl.BlockSpec((N0, D0), lambda i, pid_ref: (0, 0))]
        + [pl.BlockSpec((1, 1, D1), w1_map(g)) for g in range(G)],
        out_specs=pl.BlockSpec((G, N0, D0 + D1), lambda i, pid_ref: (i, 0, 0)),
    )
    out = pl.pallas_call(
        _body,
        grid_spec=grid_spec,
        out_shape=jax.ShapeDtypeStruct((B, N0, D0 + D1), jnp.float32),
    )(pid, w0_2d, *([w1_3d] * G))
    return out.reshape(*position_ids.shape, N0, D0 + D1)


# speedup vs baseline: 2.9876x; 1.2810x over previous
import jax
import jax.numpy as jnp
from jax.experimental import pallas as pl
from jax.experimental.pallas import tpu as pltpu

N0, N1 = 64, 64
D0, D1 = 1024, 1024
G = 8


def _body(pid_ref, w0_ref, *refs):
    w1_refs, out_ref = refs[:-1], refs[-1]
    for g in range(G):
        out_ref[g, :, :D0] = w0_ref[...]
        out_ref[g, :, D0:] = jnp.broadcast_to(w1_refs[g][0], (N0, D1))


def kernel(position_ids, w0, w1):
    B = position_ids.size
    pid = position_ids.reshape(-1).astype(jnp.int32)
    w0_2d = w0.reshape(N0, D0)
    w1_3d = w1.reshape(N1, 1, D1)

    def w1_map(g):
        return lambda i, pid_ref: (pid_ref[i * G + g], 0, 0)

    grid_spec = pltpu.PrefetchScalarGridSpec(
        num_scalar_prefetch=1,
        grid=(B // G,),
        in_specs=[pl.BlockSpec((N0, D0), lambda i, pid_ref: (0, 0))]
        + [pl.BlockSpec((1, 1, D1), w1_map(g)) for g in range(G)],
        out_specs=pl.BlockSpec((G, N0, D0 + D1), lambda i, pid_ref: (i, 0, 0)),
    )
    out = pl.pallas_call(
        _body,
        grid_spec=grid_spec,
        out_shape=jax.ShapeDtypeStruct((B, N0, D0 + D1), jnp.float32),
    )(pid, w0_2d, *([w1_3d] * G))
    return out.reshape(*position_ids.shape, N0, D0 + D1)


# speedup vs baseline: 3.0770x; 1.0299x over previous
import jax
import jax.numpy as jnp
from jax.experimental import pallas as pl
from jax.experimental.pallas import tpu as pltpu

N0, N1 = 64, 64
D0, D1 = 1024, 1024
G = 16


def _body(pid_ref, w0_ref, *refs):
    w1_refs, out_ref = refs[:-1], refs[-1]
    for g in range(G):
        out_ref[g, :, :D0] = w0_ref[...]
        out_ref[g, :, D0:] = jnp.broadcast_to(w1_refs[g][0], (N0, D1))


def kernel(position_ids, w0, w1):
    B = position_ids.size
    pid = position_ids.reshape(-1).astype(jnp.int32)
    w0_2d = w0.reshape(N0, D0)
    w1_3d = w1.reshape(N1, 1, D1)

    def w1_map(g):
        return lambda i, pid_ref: (pid_ref[i * G + g], 0, 0)

    grid_spec = pltpu.PrefetchScalarGridSpec(
        num_scalar_prefetch=1,
        grid=(B // G,),
        in_specs=[pl.BlockSpec((N0, D0), lambda i, pid_ref: (0, 0))]
        + [pl.BlockSpec((1, 1, D1), w1_map(g)) for g in range(G)],
        out_specs=pl.BlockSpec((G, N0, D0 + D1), lambda i, pid_ref: (i, 0, 0)),
    )
    out = pl.pallas_call(
        _body,
        grid_spec=grid_spec,
        out_shape=jax.ShapeDtypeStruct((B, N0, D0 + D1), jnp.float32),
    )(pid, w0_2d, *([w1_3d] * G))
    return out.reshape(*position_ids.shape, N0, D0 + D1)
